# baseline (speedup 1.0000x reference)
import jax
import jax.numpy as jnp
from jax.experimental import pallas as pl
from jax.experimental.pallas import tpu as pltpu


def _fused_kernel(x_ref, w1_ref, b1_ref, w2_ref, b2_ref, o_ref):
    C = x_ref.shape[1]
    y = b1_ref[...]
    for c in range(C):
        y = y + jnp.dot(x_ref[:, c, :].astype(jnp.bfloat16),
                        w1_ref[c].astype(jnp.bfloat16),
                        preferred_element_type=jnp.float32)
    z = jnp.dot(y.astype(jnp.bfloat16), w2_ref[...].astype(jnp.bfloat16),
                preferred_element_type=jnp.float32)
    o_ref[...] = (z + b2_ref[...]).astype(o_ref.dtype)


def kernel(x, bb_w_t, bb_b, heads_w_t, heads_b):
    B, C, H, W = x.shape
    K = C * H * W
    HW = H * W
    Kp, Nb = bb_w_t.shape
    Nb2, Nh = heads_w_t.shape
    assert Nb == Nb2
    assert K == Kp, (K, Kp)

    xv = x.reshape(B, C, HW)
    w1v = bb_w_t.reshape(C, HW, Nb)

    M = B
    tm = 512
    while M % tm and tm > 8:
        tm //= 2
    Mp = ((M + tm - 1) // tm) * tm
    if Mp != M:
        xv = jnp.pad(xv, ((0, Mp - M), (0, 0), (0, 0)))

    grid = (Mp // tm,)
    out = pl.pallas_call(
        _fused_kernel,
        out_shape=jax.ShapeDtypeStruct((Mp, Nh), jnp.float32),
        grid=grid,
        in_specs=[
            pl.BlockSpec((tm, C, HW), lambda i: (i, 0, 0)),
            pl.BlockSpec((C, HW, Nb), lambda i: (0, 0, 0)),
            pl.BlockSpec((1, Nb), lambda i: (0, 0)),
            pl.BlockSpec((Nb, Nh), lambda i: (0, 0)),
            pl.BlockSpec((1, Nh), lambda i: (0, 0)),
        ],
        out_specs=pl.BlockSpec((tm, Nh), lambda i: (i, 0)),
        compiler_params=pltpu.CompilerParams(
            dimension_semantics=("parallel",),
            vmem_limit_bytes=48 * 1024 * 1024,
        ),
        cost_estimate=pl.CostEstimate(
            flops=2 * Mp * Kp * Nb + 2 * Mp * Nb * Nh,
            transcendentals=0,
            bytes_accessed=4 * (Mp * Kp + Kp * Nb + Nb * Nh + Mp * Nh),
        ),
    )(xv, w1v, bb_b, heads_w_t, heads_b)

    out = out[:M]
    return [out]


# speedup vs baseline: 1.9761x; 1.9761x over previous
import jax
import jax.numpy as jnp
from jax.experimental import pallas as pl
from jax.experimental.pallas import tpu as pltpu


def _fused_kernel(x_ref, w1_ref, b1_ref, w2_ref, b2_ref, o_ref):
    y = jnp.dot(x_ref[...].astype(jnp.bfloat16),
                w1_ref[...].astype(jnp.bfloat16),
                preferred_element_type=jnp.float32)
    y = y + b1_ref[...]
    z = jnp.dot(y.astype(jnp.bfloat16),
                w2_ref[...].astype(jnp.bfloat16),
                preferred_element_type=jnp.float32)
    o_ref[...] = (z + b2_ref[...]).astype(o_ref.dtype)


def kernel(x, bb_w_t, bb_b, heads_w_t, heads_b):
    B = x.shape[0]
    xf = x.reshape(B, -1)
    M, K = xf.shape
    Kp, Nb = bb_w_t.shape
    Nb2, Nh = heads_w_t.shape
    assert Nb == Nb2

    if K != Kp:
        xf = jnp.pad(xf, ((0, 0), (0, Kp - K)))

    tm = 1024
    while M % tm and tm > 8:
        tm //= 2
    Mp = ((M + tm - 1) // tm) * tm
    if Mp != M:
        xf = jnp.pad(xf, ((0, Mp - M), (0, 0)))

    grid = (Mp // tm,)
    out = pl.pallas_call(
        _fused_kernel,
        out_shape=jax.ShapeDtypeStruct((Mp, Nh), jnp.float32),
        grid=grid,
        in_specs=[
            pl.BlockSpec((tm, Kp), lambda i: (i, 0)),
            pl.BlockSpec((Kp, Nb), lambda i: (0, 0)),
            pl.BlockSpec((1, Nb), lambda i: (0, 0)),
            pl.BlockSpec((Nb, Nh), lambda i: (0, 0)),
            pl.BlockSpec((1, Nh), lambda i: (0, 0)),
        ],
        out_specs=pl.BlockSpec((tm, Nh), lambda i: (i, 0)),
        compiler_params=pltpu.CompilerParams(
            dimension_semantics=("parallel",),
            vmem_limit_bytes=60 * 1024 * 1024,
        ),
        cost_estimate=pl.CostEstimate(
            flops=2 * Mp * Kp * Nb + 2 * Mp * Nb * Nh,
            transcendentals=0,
            bytes_accessed=4 * (Mp * Kp + Kp * Nb + Nb * Nh + Mp * Nh),
        ),
    )(xf, bb_w_t, bb_b, heads_w_t, heads_b)

    out = out[:M]
    return [out]
